# Initial kernel scaffold; baseline (speedup 1.0000x reference)
import jax
import jax.numpy as jnp
from jax.experimental import pallas as pl


def kernel(x, vocab):
    raise NotImplementedError("write your pallas kernel here")


# speedup vs baseline: 1.1138x; 1.1138x over previous
import functools

import jax
import jax.numpy as jnp
from jax import lax
from jax.experimental import pallas as pl
from jax.experimental.pallas import tpu as pltpu
from jax.experimental.pallas import tpu_sc as plsc

VOCAB = 1_000_000
D = 32
B = 16384 * 50
NC, NS = 2, 16
NW = NC * NS
B_PER_W = B // NW
CHUNK = 1280
N_CHUNKS = B_PER_W // CHUNK


def _body(idx_hbm, table_hbm, out_hbm, idx_v, rows0, rows1, sem0, sem1):
    wid = lax.axis_index("s") * NC + lax.axis_index("c")
    base = wid * B_PER_W
    pltpu.sync_copy(idx_hbm.at[pl.ds(base, B_PER_W)], idx_v)

    rows = (rows0, rows1)
    sems = (sem0, sem1)

    def gather(c, b):
        return pltpu.make_async_copy(
            table_hbm.at[idx_v.at[pl.ds(c * CHUNK, CHUNK)]], rows[b], sems[b])

    gather(0, 0).start()
    gather(1, 1).start()

    @pl.loop(0, N_CHUNKS - 2, step=2)
    def _(c):
        for b in range(2):
            cc = c + b
            gather(cc, b).wait()
            pltpu.sync_copy(rows[b], out_hbm.at[pl.ds(base + cc * CHUNK, CHUNK)])
            gather(cc + 2, b).start()

    for b in range(2):
        cc = N_CHUNKS - 2 + b
        gather(cc, b).wait()
        pltpu.sync_copy(rows[b], out_hbm.at[pl.ds(base + cc * CHUNK, CHUNK)])


@functools.partial(jax.jit, static_argnames=())
def _embed(idx_flat, table):
    mesh = plsc.VectorSubcoreMesh(
        core_axis_name="c", subcore_axis_name="s", num_cores=NC, num_subcores=NS)
    k = pl.kernel(
        _body,
        out_type=jax.ShapeDtypeStruct((B, D), jnp.float32),
        mesh=mesh,
        scratch_types=[
            pltpu.VMEM((B_PER_W,), jnp.int32),
            pltpu.VMEM((CHUNK, D), jnp.float32),
            pltpu.VMEM((CHUNK, D), jnp.float32),
            pltpu.SemaphoreType.DMA,
            pltpu.SemaphoreType.DMA,
        ],
        compiler_params=pltpu.CompilerParams(use_tc_tiling_on_sc=False),
    )
    return k(idx_flat, table)


def kernel(x, vocab):
    idx_flat = x.reshape(-1).astype(jnp.int32)
    out = _embed(idx_flat, vocab)
    return out.reshape(x.shape + (D,))
